# Initial kernel scaffold; baseline (speedup 1.0000x reference)
import jax
import jax.numpy as jnp
from jax.experimental import pallas as pl


def kernel(x, edge_index, W1a, b1a, W1b, b1b, W2a, b2a, W2b, b2b, W3a, b3a, W3b, b3b, Wl, bl):
    raise NotImplementedError("write your pallas kernel here")


# speedup vs baseline: 3.6326x; 3.6326x over previous
import functools

import jax
import jax.numpy as jnp
from jax import lax
from jax.experimental import pallas as pl
from jax.experimental.pallas import tpu as pltpu
from jax.experimental.pallas import tpu_sc as plsc

N = 10000
NP = 10240
E = 320000
BATCH = 128
NBATCH = 158
NPAIR = NBATCH // 2
EPT = NBATCH * BATCH
EP = EPT * 16
FP = 384
RB = 512
GRID = NP // RB


def _make_agg(hf, npass):
    mesh = plsc.VectorSubcoreMesh(core_axis_name="c", subcore_axis_name="s")

    @functools.partial(
        pl.kernel,
        out_type=jax.ShapeDtypeStruct((NP, 2 * npass, hf), jnp.float32),
        mesh=mesh,
        compiler_params=pltpu.CompilerParams(use_tc_tiling_on_sc=False),
        scratch_types=[
            pltpu.VMEM_SHARED((NP, hf), jnp.float32),
            pltpu.VMEM((NBATCH, BATCH), jnp.int32),
            pltpu.VMEM((NBATCH, BATCH), jnp.int32),
            pltpu.VMEM((BATCH, hf), jnp.float32),
            pltpu.VMEM((BATCH, hf), jnp.float32),
            pltpu.SemaphoreType.DMA,
            pltpu.SemaphoreType.DMA,
        ],
    )
    def agg(h, srck, dst, zeros, out, acc, gid, sid, rows0, rows1,
            sem0, sem1):
        c = lax.axis_index("c")
        s = lax.axis_index("s")
        pltpu.sync_copy(dst.at[s], sid)
        for q in range(npass):
            k = c * npass + q
            pltpu.sync_copy(zeros, acc.at[pl.ds(s * 640, 640)])
            pltpu.sync_copy(srck.at[k, s], gid)
            plsc.subcore_barrier()

            pltpu.async_copy(h.at[gid.at[0]], rows0, sem0)

            def pair(j, carry):
                b = 2 * j
                pltpu.async_copy(h.at[gid.at[b + 1]], rows1, sem1)
                pltpu.make_async_copy(h.at[gid.at[b]], rows0, sem0).wait()
                pltpu.sync_copy(rows0, acc.at[sid.at[b]], add=True)

                @pl.when(j < NPAIR - 1)
                def _():
                    pltpu.async_copy(h.at[gid.at[b + 2]], rows0, sem0)

                pltpu.make_async_copy(
                    h.at[gid.at[b + 1]], rows1, sem1).wait()
                pltpu.sync_copy(rows1, acc.at[sid.at[b + 1]], add=True)
                return carry

            lax.fori_loop(0, NPAIR, pair, 0, unroll=False)
            plsc.subcore_barrier()
            pltpu.sync_copy(acc.at[pl.ds(s * 640, 640)],
                            out.at[pl.ds(s * 640, 640), k])
            if q + 1 < npass:
                plsc.subcore_barrier()

    return agg


_agg_d = _make_agg(64, 1)
_agg_h = _make_agg(96, 2)


def _row_mask(o):
    rows = pl.program_id(0) * RB + lax.broadcasted_iota(
        jnp.int32, o.shape, 0)
    return jnp.where(rows < N, o, 0.0)


def _mlp_body(x_ref, a_ref, wa_ref, ba_ref, wb_ref, bb_ref, o_ref):
    h = x_ref[...] + a_ref[...]
    t = jnp.maximum(
        jnp.dot(h, wa_ref[...], preferred_element_type=jnp.float32)
        + ba_ref[...], 0.0)
    o = jnp.maximum(
        jnp.dot(t, wb_ref[...], preferred_element_type=jnp.float32)
        + bb_ref[...], 0.0)
    o_ref[...] = _row_mask(o)


def _mlp(x, agg, wa, ba, wb, bb):
    fin = x.shape[1]
    return pl.pallas_call(
        _mlp_body,
        grid=(GRID,),
        in_specs=[
            pl.BlockSpec((RB, fin), lambda i: (i, 0)),
            pl.BlockSpec((RB, fin), lambda i: (i, 0)),
            pl.BlockSpec((fin, FP), lambda i: (0, 0)),
            pl.BlockSpec((1, FP), lambda i: (0, 0)),
            pl.BlockSpec((FP, FP), lambda i: (0, 0)),
            pl.BlockSpec((1, FP), lambda i: (0, 0)),
        ],
        out_specs=pl.BlockSpec((RB, FP), lambda i: (i, 0)),
        out_shape=jax.ShapeDtypeStruct((NP, FP), jnp.float32),
    )(x, agg, wa, ba, wb, bb)


def _mlp_final_body(x_ref, a_ref, wa_ref, ba_ref, wb_ref, bb_ref,
                    wl_ref, bl_ref, o_ref, acc_ref):
    i = pl.program_id(0)
    h = x_ref[...] + a_ref[...]
    t = jnp.maximum(
        jnp.dot(h, wa_ref[...], preferred_element_type=jnp.float32)
        + ba_ref[...], 0.0)
    o = jnp.maximum(
        jnp.dot(t, wb_ref[...], preferred_element_type=jnp.float32)
        + bb_ref[...], 0.0)
    o = _row_mask(o)
    part = jnp.sum(o, axis=0, keepdims=True)

    @pl.when(i == 0)
    def _():
        acc_ref[...] = jnp.zeros_like(acc_ref)

    acc_ref[0:1, :] = acc_ref[0:1, :] + part

    @pl.when(i == GRID - 1)
    def _():
        pooled = acc_ref[0:1, :]
        logit = jnp.sum(pooled * wl_ref[...]) + bl_ref[0, 0]
        sig = 1.0 / (1.0 + jnp.exp(-logit))
        o_ref[...] = jnp.stack(
            [1.0 - sig, sig]).reshape(1, 2).astype(jnp.float32)


def _mlp_final(x, agg, wa, ba, wb, bb, wlt, bl2):
    return pl.pallas_call(
        _mlp_final_body,
        grid=(GRID,),
        in_specs=[
            pl.BlockSpec((RB, FP), lambda i: (i, 0)),
            pl.BlockSpec((RB, FP), lambda i: (i, 0)),
            pl.BlockSpec((FP, FP), lambda i: (0, 0)),
            pl.BlockSpec((1, FP), lambda i: (0, 0)),
            pl.BlockSpec((FP, FP), lambda i: (0, 0)),
            pl.BlockSpec((1, FP), lambda i: (0, 0)),
            pl.BlockSpec((1, FP), lambda i: (0, 0)),
            pl.BlockSpec((1, 1), lambda i: (0, 0)),
        ],
        out_specs=pl.BlockSpec((1, 2), lambda i: (0, 0)),
        out_shape=jax.ShapeDtypeStruct((1, 2), jnp.float32),
        scratch_shapes=[pltpu.VMEM((8, FP), jnp.float32)],
    )(x, agg, wa, ba, wb, bb, wlt, bl2)


def _pad_w(w):
    fi, fo = w.shape
    return jnp.pad(w, ((0, FP - fi), (0, FP - fo)))


def _pad_b(b):
    return jnp.pad(b, (0, FP - b.shape[0])).reshape(1, FP)


def _edge_ids(src, dst, nchunk):
    src_p = jnp.pad(src, (0, EP - E))
    dst_p = jnp.pad(dst, (0, EP - E), constant_values=N)
    srck = jnp.stack([nchunk * src_p + k for k in range(nchunk)])
    return (srck.reshape(nchunk, 16, NBATCH, BATCH),
            dst_p.reshape(16, NBATCH, BATCH))


def kernel(x, edge_index, W1a, b1a, W1b, b1b, W2a, b2a, W2b, b2b,
           W3a, b3a, W3b, b3b, Wl, bl):
    src = edge_index[0]
    dst = edge_index[1]
    src2, dst_r = _edge_ids(src, dst, 2)
    src4, _ = _edge_ids(src, dst, 4)

    zeros_d = jnp.zeros((640, 64), jnp.float32)
    zeros_h = jnp.zeros((640, 96), jnp.float32)

    x_p = jnp.pad(x, ((0, NP - N), (0, 0)))
    w1a = jnp.pad(W1a, ((0, 0), (0, FP - W1a.shape[1])))
    w1b, w2a, w2b, w3a, w3b = map(_pad_w, (W1b, W2a, W2b, W3a, W3b))
    bb1a, bb1b, bb2a, bb2b, bb3a, bb3b = map(
        _pad_b, (b1a, b1b, b2a, b2b, b3a, b3b))
    wlt = jnp.pad(Wl[:, 0], (0, FP - Wl.shape[0])).reshape(1, FP)
    bl2 = bl.reshape(1, 1)

    agg1 = _agg_d(x_p.reshape(2 * NP, 64), src2, dst_r, zeros_d)
    h1 = _mlp(x_p, agg1.reshape(NP, 128), w1a, bb1a, w1b, bb1b)

    agg2 = _agg_h(h1.reshape(4 * NP, 96), src4, dst_r, zeros_h)
    h2 = _mlp(h1, agg2.reshape(NP, FP), w2a, bb2a, w2b, bb2b)

    agg3 = _agg_h(h2.reshape(4 * NP, 96), src4, dst_r, zeros_h)
    return _mlp_final(h2, agg3.reshape(NP, FP), w3a, bb3a, w3b, bb3b,
                      wlt, bl2)
